# baseline (speedup 1.0000x reference)
import functools

import jax
import jax.numpy as jnp
from jax import lax
from jax.experimental import pallas as pl
from jax.experimental.pallas import tpu as pltpu
from jax.experimental.pallas import tpu_sc as plsc

N = 100000
NPAD = 100096
ROWS = 782
M = 64
K = 1024
L = 6
PADVAL = 1e9


def _fps_select_kernel(px_ref, py_ref, c_ref, tj_ref, pos_ref,
                       dists_ref, idx_ref):
    i = pl.program_id(0)

    @pl.when(i == 0)
    def _():
        dists_ref[...] = jnp.full((ROWS, 128), jnp.inf, jnp.float32)
        idx_ref[0] = jnp.int32(0)

    px = px_ref[...]
    py = py_ref[...]
    lin = (lax.broadcasted_iota(jnp.int32, (ROWS, 128), 0) * 128
           + lax.broadcasted_iota(jnp.int32, (ROWS, 128), 1))
    cur = idx_ref[0]
    sel = (lin == cur)
    cx = jnp.sum(jnp.where(sel, px, 0.0))
    cy = jnp.sum(jnp.where(sel, py, 0.0))

    dx = px - cx
    dy = py - cy
    d2 = dx * dx + dy * dy
    crow = jnp.concatenate(
        [jnp.reshape(cx, (1, 1)), jnp.reshape(cy, (1, 1))], axis=1)
    c_ref[pl.ds(i, 1), :] = crow

    pad = lin >= N
    nd = jnp.where(pad, -1.0, jnp.minimum(dists_ref[...], d2))
    dists_ref[...] = nd
    mx = jnp.max(nd)
    nxt = jnp.min(jnp.where(nd == mx, lin, jnp.int32(2 ** 30)))
    idx_ref[0] = nxt

    d2b = lax.bitcast_convert_type(d2, jnp.int32)

    def bs_body(_, lohi):
        lo, hi = lohi
        mid = lo + (hi - lo) // 2
        cnt = jnp.sum((d2b <= mid).astype(jnp.int32))
        good = cnt >= K
        return (jnp.where(good, lo, mid + 1), jnp.where(good, mid, hi))

    lo, hi = lax.fori_loop(0, 31, bs_body,
                           (jnp.int32(0), jnp.int32(0x7F800000)))
    tstar = hi
    cnt_lt = jnp.sum((d2b < tstar).astype(jnp.int32))
    eq = (d2b == tstar)

    def js_body(_, lohi):
        lo, hi = lohi
        mid = lo + (hi - lo) // 2
        cnt = cnt_lt + jnp.sum((eq & (lin <= mid)).astype(jnp.int32))
        good = cnt >= K
        return (jnp.where(good, lo, mid + 1), jnp.where(good, mid, hi))

    jlo, jhi = lax.fori_loop(0, 17, js_body,
                             (jnp.int32(0), jnp.int32(NPAD - 1)))
    tjrow = jnp.concatenate(
        [jnp.reshape(tstar, (1, 1)), jnp.reshape(jhi, (1, 1))], axis=1)
    tj_ref[pl.ds(i, 1), :] = tjrow

    mask = (d2b < tstar) | (eq & (lin <= jhi))
    mf = mask.astype(jnp.float32)
    lane = lax.broadcasted_iota(jnp.int32, (128, 128), 0)
    ut = (lane <= lax.broadcasted_iota(jnp.int32, (128, 128), 1)
          ).astype(jnp.float32)
    incl = jnp.dot(mf, ut, preferred_element_type=jnp.float32)
    rowtot = incl[:, 127:128]
    rr = lax.broadcasted_iota(jnp.int32, (ROWS, ROWS), 0)
    ls = (lax.broadcasted_iota(jnp.int32, (ROWS, ROWS), 1) < rr
          ).astype(jnp.float32)
    rowpref = jnp.dot(ls, rowtot, preferred_element_type=jnp.float32)
    posf = rowpref + incl - mf
    shift = ((i % 32) // 2) * (K + 8)
    pos_ref[0] = jnp.where(mask, posf.astype(jnp.int32), jnp.int32(K)) + shift


def _fps_select(px, py):
    return pl.pallas_call(
        _fps_select_kernel,
        grid=(M,),
        in_specs=[
            pl.BlockSpec((ROWS, 128), lambda i: (0, 0)),
            pl.BlockSpec((ROWS, 128), lambda i: (0, 0)),
        ],
        out_specs=[
            pl.BlockSpec((M, 2), lambda i: (0, 0)),
            pl.BlockSpec((M, 2), lambda i: (0, 0)),
            pl.BlockSpec((1, ROWS, 128), lambda i: (i, 0, 0)),
        ],
        out_shape=[
            jax.ShapeDtypeStruct((M, 2), jnp.float32),
            jax.ShapeDtypeStruct((M, 2), jnp.int32),
            jax.ShapeDtypeStruct((M, ROWS, 128), jnp.int32),
        ],
        scratch_shapes=[
            pltpu.VMEM((ROWS, 128), jnp.float32),
            pltpu.SMEM((1,), jnp.int32),
        ],
    )(px, py)


CHUNK = 25024
NCH = NPAD // CHUNK


def _sc_gather_kernel(pos_hbm, p16_hbm, z_hbm, out_hbm, posv, patch, stage):
    info = plsc.get_sparse_core_info()
    nc = info.num_cores
    wid = lax.axis_index("s") * nc + lax.axis_index("c")

    sid = lax.axis_index("s")
    base = sid * (K + 8)
    for t in range(2):
        c = wid + 32 * t
        pltpu.sync_copy(z_hbm, patch.at[pl.ds(base, K + 8)])
        pltpu.sync_copy(pos_hbm.at[c], posv)

        def chunk(h, carry):
            pltpu.sync_copy(p16_hbm.at[pl.ds(h * 128, 128)], stage)
            pltpu.sync_copy(stage, patch.at[posv.at[h]], add=True)
            return carry

        lax.fori_loop(0, ROWS, chunk, jnp.int32(0))
        pltpu.sync_copy(patch.at[pl.ds(base, K)], out_hbm.at[c])


def _sc_gather(pos, p16, zeros):
    mesh = plsc.VectorSubcoreMesh(core_axis_name="c", subcore_axis_name="s")
    kern = functools.partial(
        pl.kernel,
        out_type=jax.ShapeDtypeStruct((M, K, 16), jnp.float32),
        mesh=mesh,
        compiler_params=pltpu.CompilerParams(use_tc_tiling_on_sc=False),
        scratch_types=[
            pltpu.VMEM((ROWS, 128), jnp.int32),
            pltpu.VMEM_SHARED((16 * (K + 8), 16), jnp.float32),
            pltpu.VMEM((128, 16), jnp.float32),
        ],
    )(_sc_gather_kernel)
    return kern(pos, p16, zeros)


def _erf(x):
    a = jnp.abs(x)
    t = 1.0 / (1.0 + 0.3275911 * a)
    poly = t * (0.254829592 + t * (-0.284496736 + t * (1.421413741
               + t * (-1.453152027 + t * 1.061405429))))
    y = 1.0 - poly * jnp.exp(-a * a)
    return jnp.where(x < 0, -y, y)


def _gelu(x):
    return 0.5 * x * (1.0 + _erf(x * 0.7071067811865476))


def _encode_kernel(pt_ref, c_ref, w1_ref, b1_ref, w2_ref, b2_ref,
                   w3_ref, b3_ref, w4_ref, b4_ref, out_ref):
    i = pl.program_id(0)
    pt = pt_ref[0]
    crow = c_ref[pl.ds(i, 1), :]
    dx = pt[:, 0:1] - crow[:, 0:1]
    dy = pt[:, 1:2] - crow[:, 1:2]
    d2 = dx * dx + dy * dy
    r = jnp.sqrt(d2)
    pos = r > 0.0
    st = jnp.where(pos, dy / jnp.where(pos, r, 1.0), 0.0)
    ct = jnp.where(pos, dx / jnp.where(pos, r, 1.0), 1.0)

    fbits = (lax.broadcasted_iota(jnp.int32, (1, L), 1) + 127) << 23
    freqs = lax.bitcast_convert_type(fbits, jnp.float32) * jnp.pi
    xx = dx * freqs
    yy = dy * freqs
    feats = jnp.concatenate(
        [dx, dy, r, st, ct,
         jnp.sin(xx), jnp.cos(xx), jnp.sin(yy), jnp.cos(yy),
         jnp.zeros((K, 3), jnp.float32)], axis=1)

    h = _gelu(jnp.dot(feats, w1_ref[...],
                      preferred_element_type=jnp.float32) + b1_ref[...])
    h = _gelu(jnp.dot(h, w2_ref[...],
                      preferred_element_type=jnp.float32) + b2_ref[...])
    h_max = jnp.max(h, axis=0, keepdims=True)
    h_mean = jnp.sum(h, axis=0, keepdims=True) / K

    rm = jnp.sum(r) / K
    rmax = jnp.max(r)
    rmin = jnp.min(r)
    rstd = jnp.sqrt(jnp.sum((r - rm) ** 2) / (K - 1))
    stats = jnp.concatenate(
        [jnp.reshape(rm, (1, 1)), jnp.reshape(rmax, (1, 1)),
         jnp.reshape(rmin, (1, 1)), jnp.reshape(rstd, (1, 1))], axis=1)

    z = jnp.concatenate([h_max, h_mean, stats], axis=1)
    g = _gelu(jnp.dot(z, w3_ref[...],
                      preferred_element_type=jnp.float32) + b3_ref[...])
    o = jnp.dot(g, w4_ref[...],
                preferred_element_type=jnp.float32) + b4_ref[...]
    out_ref[pl.ds(i, 1), :] = o


def _encode(patches, C, w1e, b1, w2, b2, w3, b3, w4, b4):
    full = lambda shape: pl.BlockSpec(shape, lambda i: tuple(0 for _ in shape))
    return pl.pallas_call(
        _encode_kernel,
        grid=(M,),
        in_specs=[
            pl.BlockSpec((1, K, 16), lambda i: (i, 0, 0)),
            full((M, 2)),
            full((32, 64)), full((1, 64)),
            full((64, 128)), full((1, 128)),
            full((260, 128)), full((1, 128)),
            full((128, 256)), full((1, 256)),
        ],
        out_specs=pl.BlockSpec((M, 256), lambda i: (0, 0)),
        out_shape=jax.ShapeDtypeStruct((M, 256), jnp.float32),
    )(patches, C, w1e, b1, w2, b2, w3, b3, w4, b4)


def kernel(P, W1, b1, W2, b2, W3, b3, W4, b4):
    Pp = jnp.pad(P, ((0, NPAD - N), (0, 0)), constant_values=PADVAL)
    px = Pp[:, 0].reshape(ROWS, 128)
    py = Pp[:, 1].reshape(ROWS, 128)

    C, tj, pos = _fps_select(px, py)
    del tj

    p16 = jnp.pad(P, ((0, NPAD - N), (0, 14)))
    zeros = jnp.zeros((K + 8, 16), jnp.float32)

    patches = _sc_gather(pos, p16, zeros)

    w1e = jnp.pad(W1, ((0, 3), (0, 0)))
    F_out = _encode(patches, C, w1e, b1[None], W2, b2[None],
                    W3, b3[None], W4, b4[None])
    return (F_out, C)


# speedup vs baseline: 1.1379x; 1.1379x over previous
import functools

import jax
import jax.numpy as jnp
from jax import lax
from jax.experimental import pallas as pl
from jax.experimental.pallas import tpu as pltpu
from jax.experimental.pallas import tpu_sc as plsc

N = 100000
NPAD = 100096
ROWS = 782
M = 64
K = 1024
L = 6
PADVAL = 1e9


def _fps_select_kernel(px_ref, py_ref, c_ref, tj_ref, pos_ref,
                       dists_ref, idx_ref):
    i = pl.program_id(0)

    @pl.when(i == 0)
    def _():
        dists_ref[...] = jnp.full((ROWS, 128), jnp.inf, jnp.float32)
        idx_ref[0] = jnp.int32(0)

    px = px_ref[...]
    py = py_ref[...]
    lin = (lax.broadcasted_iota(jnp.int32, (ROWS, 128), 0) * 128
           + lax.broadcasted_iota(jnp.int32, (ROWS, 128), 1))
    cur = idx_ref[0]
    sel = (lin == cur)
    cx = jnp.sum(jnp.where(sel, px, 0.0))
    cy = jnp.sum(jnp.where(sel, py, 0.0))

    dx = px - cx
    dy = py - cy
    d2 = dx * dx + dy * dy
    crow = jnp.concatenate(
        [jnp.reshape(cx, (1, 1)), jnp.reshape(cy, (1, 1))], axis=1)
    c_ref[pl.ds(i, 1), :] = crow

    pad = lin >= N
    nd = jnp.where(pad, -1.0, jnp.minimum(dists_ref[...], d2))
    dists_ref[...] = nd
    mx = jnp.max(nd)
    nxt = jnp.min(jnp.where(nd == mx, lin, jnp.int32(2 ** 30)))
    idx_ref[0] = nxt

    d2b = lax.bitcast_convert_type(d2, jnp.int32)

    def bs_body(_, lohi):
        lo, hi = lohi
        mid = lo + (hi - lo) // 2
        cnt = jnp.sum((d2b <= mid).astype(jnp.int32))
        good = cnt >= K
        return (jnp.where(good, lo, mid + 1), jnp.where(good, mid, hi))

    lo, hi = lax.fori_loop(0, 31, bs_body,
                           (jnp.int32(0), jnp.int32(0x7F800000)))
    tstar = hi
    cnt_lt = jnp.sum((d2b < tstar).astype(jnp.int32))
    eq = (d2b == tstar)

    def js_body(_, lohi):
        lo, hi = lohi
        mid = lo + (hi - lo) // 2
        cnt = cnt_lt + jnp.sum((eq & (lin <= mid)).astype(jnp.int32))
        good = cnt >= K
        return (jnp.where(good, lo, mid + 1), jnp.where(good, mid, hi))

    jlo, jhi = lax.fori_loop(0, 17, js_body,
                             (jnp.int32(0), jnp.int32(NPAD - 1)))
    tjrow = jnp.concatenate(
        [jnp.reshape(tstar, (1, 1)), jnp.reshape(jhi, (1, 1))], axis=1)
    tj_ref[pl.ds(i, 1), :] = tjrow

    mask = (d2b < tstar) | (eq & (lin <= jhi))
    mf = mask.astype(jnp.float32)
    lane = lax.broadcasted_iota(jnp.int32, (128, 128), 0)
    ut = (lane <= lax.broadcasted_iota(jnp.int32, (128, 128), 1)
          ).astype(jnp.float32)
    incl = jnp.dot(mf, ut, preferred_element_type=jnp.float32)
    rowtot = incl[:, 127:128]
    rr = lax.broadcasted_iota(jnp.int32, (ROWS, ROWS), 0)
    ls = (lax.broadcasted_iota(jnp.int32, (ROWS, ROWS), 1) < rr
          ).astype(jnp.float32)
    rowpref = jnp.dot(ls, rowtot, preferred_element_type=jnp.float32)
    posf = rowpref + incl - mf
    shift = ((i % 32) // 2) * (K + 8)
    pos_ref[0] = jnp.where(mask, posf.astype(jnp.int32), jnp.int32(K)) + shift


def _fps_select(px, py):
    return pl.pallas_call(
        _fps_select_kernel,
        grid=(M,),
        in_specs=[
            pl.BlockSpec((ROWS, 128), lambda i: (0, 0)),
            pl.BlockSpec((ROWS, 128), lambda i: (0, 0)),
        ],
        out_specs=[
            pl.BlockSpec((M, 2), lambda i: (0, 0)),
            pl.BlockSpec((M, 2), lambda i: (0, 0)),
            pl.BlockSpec((1, ROWS, 128), lambda i: (i, 0, 0)),
        ],
        out_shape=[
            jax.ShapeDtypeStruct((M, 2), jnp.float32),
            jax.ShapeDtypeStruct((M, 2), jnp.int32),
            jax.ShapeDtypeStruct((M, ROWS, 128), jnp.int32),
        ],
        scratch_shapes=[
            pltpu.VMEM((ROWS, 128), jnp.float32),
            pltpu.SMEM((1,), jnp.int32),
        ],
    )(px, py)


CHUNK = 25024
NCH = NPAD // CHUNK


def _sc_gather_kernel(pos_hbm, p16_hbm, z_hbm, out_hbm, posv, patch,
                      stage0, stage1, sem0, sem1):
    info = plsc.get_sparse_core_info()
    nc = info.num_cores
    wid = lax.axis_index("s") * nc + lax.axis_index("c")

    sid = lax.axis_index("s")
    base = sid * (K + 8)
    stages = (stage0, stage1)
    sems = (sem0, sem1)
    for t in range(2):
        c = wid + 32 * t
        pltpu.sync_copy(z_hbm, patch.at[pl.ds(base, K + 8)])
        pltpu.sync_copy(pos_hbm.at[c], posv)

        pltpu.async_copy(p16_hbm.at[pl.ds(0, 128)], stage0, sem0)

        def pair(h2, carry):
            for b in range(2):
                h = h2 * 2 + b
                pltpu.make_async_copy(p16_hbm.at[pl.ds(0, 128)],
                                      stages[b], sems[b]).wait()

                @pl.when(h + 1 < ROWS)
                def _():
                    pltpu.async_copy(
                        p16_hbm.at[pl.ds((h + 1) * 128, 128)],
                        stages[1 - b], sems[1 - b])
                pltpu.sync_copy(stages[b], patch.at[posv.at[h]], add=True)
            return carry

        lax.fori_loop(0, ROWS // 2, pair, jnp.int32(0))
        pltpu.sync_copy(patch.at[pl.ds(base, K)], out_hbm.at[c])


def _sc_gather(pos, p16, zeros):
    mesh = plsc.VectorSubcoreMesh(core_axis_name="c", subcore_axis_name="s")
    kern = functools.partial(
        pl.kernel,
        out_type=jax.ShapeDtypeStruct((M, K, 16), jnp.float32),
        mesh=mesh,
        compiler_params=pltpu.CompilerParams(use_tc_tiling_on_sc=False),
        scratch_types=[
            pltpu.VMEM((ROWS, 128), jnp.int32),
            pltpu.VMEM_SHARED((16 * (K + 8), 16), jnp.float32),
            pltpu.VMEM((128, 16), jnp.float32),
            pltpu.VMEM((128, 16), jnp.float32),
            pltpu.SemaphoreType.DMA,
            pltpu.SemaphoreType.DMA,
        ],
    )(_sc_gather_kernel)
    return kern(pos, p16, zeros)


def _erf(x):
    a = jnp.abs(x)
    t = 1.0 / (1.0 + 0.3275911 * a)
    poly = t * (0.254829592 + t * (-0.284496736 + t * (1.421413741
               + t * (-1.453152027 + t * 1.061405429))))
    y = 1.0 - poly * jnp.exp(-a * a)
    return jnp.where(x < 0, -y, y)


def _gelu(x):
    return 0.5 * x * (1.0 + _erf(x * 0.7071067811865476))


def _encode_kernel(pt_ref, c_ref, w1_ref, b1_ref, w2_ref, b2_ref,
                   w3_ref, b3_ref, w4_ref, b4_ref, out_ref):
    i = pl.program_id(0)
    pt = pt_ref[0]
    crow = c_ref[pl.ds(i, 1), :]
    dx = pt[:, 0:1] - crow[:, 0:1]
    dy = pt[:, 1:2] - crow[:, 1:2]
    d2 = dx * dx + dy * dy
    r = jnp.sqrt(d2)
    pos = r > 0.0
    st = jnp.where(pos, dy / jnp.where(pos, r, 1.0), 0.0)
    ct = jnp.where(pos, dx / jnp.where(pos, r, 1.0), 1.0)

    fbits = (lax.broadcasted_iota(jnp.int32, (1, L), 1) + 127) << 23
    freqs = lax.bitcast_convert_type(fbits, jnp.float32) * jnp.pi
    xx = dx * freqs
    yy = dy * freqs
    feats = jnp.concatenate(
        [dx, dy, r, st, ct,
         jnp.sin(xx), jnp.cos(xx), jnp.sin(yy), jnp.cos(yy),
         jnp.zeros((K, 3), jnp.float32)], axis=1)

    h = _gelu(jnp.dot(feats, w1_ref[...],
                      preferred_element_type=jnp.float32) + b1_ref[...])
    h = _gelu(jnp.dot(h, w2_ref[...],
                      preferred_element_type=jnp.float32) + b2_ref[...])
    h_max = jnp.max(h, axis=0, keepdims=True)
    h_mean = jnp.sum(h, axis=0, keepdims=True) / K

    rm = jnp.sum(r) / K
    rmax = jnp.max(r)
    rmin = jnp.min(r)
    rstd = jnp.sqrt(jnp.sum((r - rm) ** 2) / (K - 1))
    stats = jnp.concatenate(
        [jnp.reshape(rm, (1, 1)), jnp.reshape(rmax, (1, 1)),
         jnp.reshape(rmin, (1, 1)), jnp.reshape(rstd, (1, 1))], axis=1)

    z = jnp.concatenate([h_max, h_mean, stats], axis=1)
    g = _gelu(jnp.dot(z, w3_ref[...],
                      preferred_element_type=jnp.float32) + b3_ref[...])
    o = jnp.dot(g, w4_ref[...],
                preferred_element_type=jnp.float32) + b4_ref[...]
    out_ref[pl.ds(i, 1), :] = o


def _encode(patches, C, w1e, b1, w2, b2, w3, b3, w4, b4):
    full = lambda shape: pl.BlockSpec(shape, lambda i: tuple(0 for _ in shape))
    return pl.pallas_call(
        _encode_kernel,
        grid=(M,),
        in_specs=[
            pl.BlockSpec((1, K, 16), lambda i: (i, 0, 0)),
            full((M, 2)),
            full((32, 64)), full((1, 64)),
            full((64, 128)), full((1, 128)),
            full((260, 128)), full((1, 128)),
            full((128, 256)), full((1, 256)),
        ],
        out_specs=pl.BlockSpec((M, 256), lambda i: (0, 0)),
        out_shape=jax.ShapeDtypeStruct((M, 256), jnp.float32),
    )(patches, C, w1e, b1, w2, b2, w3, b3, w4, b4)


def kernel(P, W1, b1, W2, b2, W3, b3, W4, b4):
    Pp = jnp.pad(P, ((0, NPAD - N), (0, 0)), constant_values=PADVAL)
    px = Pp[:, 0].reshape(ROWS, 128)
    py = Pp[:, 1].reshape(ROWS, 128)

    C, tj, pos = _fps_select(px, py)
    del tj

    p16 = jnp.pad(P, ((0, NPAD - N), (0, 14)))
    zeros = jnp.zeros((K + 8, 16), jnp.float32)

    patches = _sc_gather(pos, p16, zeros)

    w1e = jnp.pad(W1, ((0, 3), (0, 0)))
    F_out = _encode(patches, C, w1e, b1[None], W2, b2[None],
                    W3, b3[None], W4, b4[None])
    return (F_out, C)
